# Initial kernel scaffold; baseline (speedup 1.0000x reference)
import jax
import jax.numpy as jnp
from jax.experimental import pallas as pl


def kernel(x, edge_index, batch, W1, b1, W2, b2, Wc, bc):
    raise NotImplementedError("write your pallas kernel here")


# speedup vs baseline: 12.7115x; 12.7115x over previous
import functools

import jax
import jax.numpy as jnp
from jax import lax
from jax.experimental import pallas as pl
from jax.experimental.pallas import tpu as pltpu
from jax.experimental.pallas import tpu_sc as plsc

N_NODES = 10000
N_EDGES = 320000
IN_DIM = 200
HIDDEN = 64
OUT_DIM = 16
NUM_GRAPHS = 64

NC = 2
NS = 16
LANES = 16
N_PAD = 10240
ROWS_PER_TILE = N_PAD // NS
CHUNK = 80


def _dinv_body(dst_hbm, out_hbm, idx_v, ones_v, buf_v, sem, acc_sh):
    cid = lax.axis_index("c")
    sid = lax.axis_index("s")

    @pl.when(cid == 0)
    def _():
        for i in range(CHUNK // LANES):
            ones_v[pl.ds(i * LANES, LANES)] = jnp.full((LANES,), 1.0, jnp.float32)

        def zfill(i, carry):
            buf_v[pl.ds(i * LANES, LANES)] = jnp.zeros((LANES,), jnp.float32)
            return carry

        lax.fori_loop(0, ROWS_PER_TILE // LANES, zfill, 0)
        pltpu.sync_copy(buf_v, acc_sh.at[pl.ds(sid * ROWS_PER_TILE, ROWS_PER_TILE)])
        plsc.subcore_barrier()

        edges_per_tile = N_EDGES // NS
        base = sid * edges_per_tile

        def step(j, carry):
            pltpu.sync_copy(dst_hbm.at[pl.ds(base + j * CHUNK, CHUNK)], idx_v)
            pltpu.sync_copy(ones_v, acc_sh.at[idx_v], add=True)
            return carry

        lax.fori_loop(0, edges_per_tile // CHUNK, step, 0)
        plsc.subcore_barrier()

        pltpu.sync_copy(acc_sh.at[pl.ds(sid * ROWS_PER_TILE, ROWS_PER_TILE)], buf_v)

        def newton(i, carry):
            d = buf_v[pl.ds(i * LANES, LANES)] + 1.0
            bits = lax.bitcast_convert_type(d, jnp.int32)
            y = lax.bitcast_convert_type(
                jnp.int32(0x5F3759DF) - (bits >> 1), jnp.float32)
            for _ in range(3):
                y = y * (1.5 - 0.5 * d * y * y)
            buf_v[pl.ds(i * LANES, LANES)] = y
            return carry

        lax.fori_loop(0, ROWS_PER_TILE // LANES, newton, 0)
        pltpu.sync_copy(buf_v, out_hbm.at[pl.ds(sid * ROWS_PER_TILE, ROWS_PER_TILE)])


def _agg_body(u_hbm, src_hbm, dst_hbm, out_hbm, sidx_v, didx_v, rows_v, stage_v,
              sem, acc_sh):
    cid = lax.axis_index("c")
    sid = lax.axis_index("s")
    wid = cid * NS + sid

    def zfill(r, carry):
        for c in range(HIDDEN // LANES):
            stage_v[r, pl.ds(c * LANES, LANES)] = jnp.zeros((LANES,), jnp.float32)
        return carry

    lax.fori_loop(0, ROWS_PER_TILE, zfill, 0)
    pltpu.sync_copy(stage_v, acc_sh.at[pl.ds(sid * ROWS_PER_TILE, ROWS_PER_TILE), :])
    plsc.subcore_barrier()

    edges_per_worker = N_EDGES // (NC * NS)
    base = wid * edges_per_worker

    def step(j, carry):
        off = base + j * CHUNK
        pltpu.sync_copy(src_hbm.at[pl.ds(off, CHUNK)], sidx_v)
        pltpu.sync_copy(dst_hbm.at[pl.ds(off, CHUNK)], didx_v)
        pltpu.async_copy(u_hbm.at[sidx_v], rows_v, sem).wait()
        pltpu.sync_copy(rows_v, acc_sh.at[didx_v], add=True)
        return carry

    lax.fori_loop(0, edges_per_worker // CHUNK, step, 0)
    plsc.subcore_barrier()

    pltpu.sync_copy(acc_sh.at[pl.ds(sid * ROWS_PER_TILE, ROWS_PER_TILE), :],
                    out_hbm.at[cid, pl.ds(sid * ROWS_PER_TILE, ROWS_PER_TILE), :])


def _sc_mesh():
    return plsc.VectorSubcoreMesh(core_axis_name="c", subcore_axis_name="s",
                                  num_cores=NC, num_subcores=NS)


_SC_PARAMS = pltpu.CompilerParams(use_tc_tiling_on_sc=False)


def _dinv_call(dst):
    k = pl.kernel(
        _dinv_body,
        out_type=jax.ShapeDtypeStruct((N_PAD,), jnp.float32),
        mesh=_sc_mesh(),
        compiler_params=_SC_PARAMS,
        scratch_types=[
            pltpu.VMEM((CHUNK,), jnp.int32),
            pltpu.VMEM((CHUNK,), jnp.float32),
            pltpu.VMEM((ROWS_PER_TILE,), jnp.float32),
            pltpu.SemaphoreType.DMA,
            pltpu.VMEM_SHARED((N_PAD,), jnp.float32),
        ],
    )
    return k(dst)


def _agg_call(u, src, dst):
    k = pl.kernel(
        _agg_body,
        out_type=jax.ShapeDtypeStruct((NC, N_PAD, HIDDEN), jnp.float32),
        mesh=_sc_mesh(),
        compiler_params=_SC_PARAMS,
        scratch_types=[
            pltpu.VMEM((CHUNK,), jnp.int32),
            pltpu.VMEM((CHUNK,), jnp.int32),
            pltpu.VMEM((CHUNK, HIDDEN), jnp.float32),
            pltpu.VMEM((ROWS_PER_TILE, HIDDEN), jnp.float32),
            pltpu.SemaphoreType.DMA,
            pltpu.VMEM_SHARED((N_PAD, HIDDEN), jnp.float32),
        ],
    )
    return k(u, src, dst)


def _tc1_body(x_ref, w_ref, dinv_ref, u_ref):
    xw = jnp.dot(x_ref[...], w_ref[...], preferred_element_type=jnp.float32)
    u_ref[...] = xw * dinv_ref[...]


def _tc2_body(p_ref, u_ref, dinv_ref, b_ref, w_ref, u2_ref):
    agg = p_ref[0][:N_NODES] + p_ref[1][:N_NODES] + u_ref[...]
    h = jnp.maximum(dinv_ref[...] * agg + b_ref[...], 0.0)
    u2_ref[...] = jnp.dot(h, w_ref[...],
                          preferred_element_type=jnp.float32) * dinv_ref[...]


def _tc3_body(p_ref, u_ref, dinv_ref, b_ref, batch_ref, wc_ref, bc_ref, out_ref):
    agg = p_ref[0][:N_NODES] + p_ref[1][:N_NODES] + u_ref[...]
    h = jnp.maximum(dinv_ref[...] * agg + b_ref[...], 0.0)
    gid = lax.broadcasted_iota(jnp.int32, (NUM_GRAPHS, N_NODES), 0)
    onehot_t = (batch_ref[...] == gid).astype(jnp.float32)
    seg = jnp.dot(onehot_t, h, preferred_element_type=jnp.float32)
    counts = jnp.sum(onehot_t, axis=1, keepdims=True)
    hg = seg / jnp.maximum(counts, 1.0)
    out_ref[...] = jnp.dot(hg, wc_ref[...],
                           preferred_element_type=jnp.float32) + bc_ref[...]


def kernel(x, edge_index, batch, W1, b1, W2, b2, Wc, bc):
    src = edge_index[0]
    dst = edge_index[1]

    dinv_pad = _dinv_call(dst)
    dinv = dinv_pad[:N_NODES, None]

    u1 = pl.pallas_call(
        _tc1_body,
        out_shape=jax.ShapeDtypeStruct((N_NODES, HIDDEN), jnp.float32),
    )(x, W1, dinv)

    p1 = _agg_call(u1, src, dst)

    u2 = pl.pallas_call(
        _tc2_body,
        out_shape=jax.ShapeDtypeStruct((N_NODES, HIDDEN), jnp.float32),
    )(p1, u1, dinv, b1.reshape(1, HIDDEN), W2)

    p2 = _agg_call(u2, src, dst)

    out = pl.pallas_call(
        _tc3_body,
        out_shape=jax.ShapeDtypeStruct((NUM_GRAPHS, OUT_DIM), jnp.float32),
    )(p2, u2, dinv, b2.reshape(1, HIDDEN), batch.reshape(1, N_NODES),
      Wc, bc.reshape(1, OUT_DIM))
    return out
